# baseline (speedup 1.0000x reference)
import jax
import jax.numpy as jnp
from jax import lax
from jax.experimental import pallas as pl
from jax.experimental.pallas import tpu as pltpu
from jax.experimental.pallas import tpu_sc as plsc

_B, _L, _D = 4096, 50, 128
_VOCAB = 100000
_NE = 32
_NF = 32
_NPOS = _B * _L
_NWORK = 32
_PW = _NPOS // _NWORK
_P = 256
_NCHUNK = _PW // _P
_NPAIR = (_NCHUNK - 1) // 2
_NG = _P // 16
_G = 32
_GSH = 5
_NR = _P // _G


def _body(cidx_hbm, msk_hbm, ents_hbm, facts_hbm, word_hbm, pad_hbm, out_hbm,
          cidx_v, msk_v, pad_v, *sets):
    sa, sb = sets[:13], sets[13:]
    wid = lax.axis_index("s") * 2 + lax.axis_index("c")
    wbase = wid * _PW
    lane = jnp.arange(16, dtype=jnp.int32)
    c50 = jnp.full((16,), 50, dtype=jnp.int32)

    pltpu.sync_copy(cidx_hbm.at[pl.ds(wbase, _PW)], cidx_v)
    pltpu.sync_copy(msk_hbm.at[pl.ds(wbase, _PW)], msk_v)
    pltpu.sync_copy(pad_hbm, pad_v)
    padv = pad_v[...]

    def compact(srcl, dst2, cnt, srcv, posv, m):
        s = plsc.cumsum(jnp.where(m, 1, 0).astype(jnp.int32))
        lin = cnt + s - 1
        plsc.store_scatter(srcl, [lin], srcv, mask=m)
        plsc.store_scatter(dst2, [lin >> _GSH, lin & (_G - 1)], posv, mask=m)
        return cnt + jnp.sum(jnp.where(m, 1, 0).astype(jnp.int32))

    def pad_tail(srcl, dst2, cnt):
        last = jnp.maximum(cnt - 1, 0)
        lv = plsc.load_gather(srcl, [jnp.full((16,), last, dtype=jnp.int32)])
        dv = plsc.load_gather(
            dst2, [jnp.full((16,), last >> _GSH, dtype=jnp.int32),
                   jnp.full((16,), last & (_G - 1), dtype=jnp.int32)])
        for o in range(0, _G, 16):
            lin = cnt + o + lane
            plsc.store_scatter(srcl, [lin], lv)
            plsc.store_scatter(dst2, [lin >> _GSH, lin & (_G - 1)], dv)

    def prep_fire(c, S):
        (srcw, srce, srcf, dstw, dste, dstf, rows,
         semw, seme, semf, _ssw, _sse, _ssf) = S
        cbase = c * _P
        cw = jnp.int32(0)
        ce = jnp.int32(0)
        cf = jnp.int32(0)
        for g in range(_NG):
            off = cbase + g * 16
            iv = cidx_v[pl.ds(off, 16)]
            mv = msk_v[pl.ds(off, 16)]
            pos = (wbase + off) + lane
            rowb = lax.div(pos, c50) * _NE
            te = iv - _VOCAB
            cei = jnp.where((te < 0) | (te >= _NE), _NE - 1, te)
            tf = te - _NE
            cfi = jnp.where((tf < 0) | (tf >= _NF), _NF - 1, tf)
            wv = jnp.where(iv >= _VOCAB, padv, iv)
            m_e = mv == 1
            m_f = mv == 2
            m_w = ~(m_e | m_f)
            cw = compact(srcw, dstw, cw, wv, pos, m_w)
            ce = compact(srce, dste, ce, rowb + cei, pos, m_e)
            cf = compact(srcf, dstf, cf, rowb + cfi, pos, m_f)

        pad_tail(srcw, dstw, cw)
        pad_tail(srce, dste, ce)
        pad_tail(srcf, dstf, cf)

        rw = (cw + _G - 1) >> _GSH
        re_ = (ce + _G - 1) >> _GSH
        rf = (cf + _G - 1) >> _GSH

        def fire(tab, srcl, base, sem):
            def go(r, _):
                pltpu.async_copy(tab.at[srcl.at[pl.ds(r * _G, _G)]],
                                 rows.at[pl.ds(base + r * _G, _G)], sem)
                return 0
            return go

        lax.fori_loop(0, rw, fire(word_hbm, srcw, jnp.int32(0), semw), 0)
        lax.fori_loop(0, re_, fire(ents_hbm, srce, rw * _G, seme), 0)
        lax.fori_loop(0, rf, fire(facts_hbm, srcf, (rw + re_) * _G, semf), 0)
        return rw, re_, rf

    def drain(S, r3):
        (srcw, srce, srcf, dstw, dste, dstf, rows,
         semw, seme, semf, ssw, sse, ssf) = S
        rw, re_, rf = r3
        bw = jnp.int32(0)
        be = rw * _G
        bf = (rw + re_) * _G

        def wait_gather(tab, srcl, base, sem):
            def go(r, _):
                pltpu.make_async_copy(tab.at[srcl.at[pl.ds(r * _G, _G)]],
                                      rows.at[pl.ds(base + r * _G, _G)],
                                      sem).wait()
                return 0
            return go

        def scat(dst2, base, sem):
            def go(r, _):
                pltpu.async_copy(rows.at[pl.ds(base + r * _G, _G)],
                                 out_hbm.at[dst2.at[r]], sem)
                return 0
            return go

        def wait_scat(dst2, base, sem):
            def go(r, _):
                pltpu.make_async_copy(rows.at[pl.ds(base + r * _G, _G)],
                                      out_hbm.at[dst2.at[r]], sem).wait()
                return 0
            return go

        lax.fori_loop(0, rw, wait_gather(word_hbm, srcw, bw, semw), 0)
        lax.fori_loop(0, rw, scat(dstw, bw, ssw), 0)
        lax.fori_loop(0, re_, wait_gather(ents_hbm, srce, be, seme), 0)
        lax.fori_loop(0, re_, scat(dste, be, sse), 0)
        lax.fori_loop(0, rf, wait_gather(facts_hbm, srcf, bf, semf), 0)
        lax.fori_loop(0, rf, scat(dstf, bf, ssf), 0)
        lax.fori_loop(0, rw, wait_scat(dstw, bw, ssw), 0)
        lax.fori_loop(0, re_, wait_scat(dste, be, sse), 0)
        lax.fori_loop(0, rf, wait_scat(dstf, bf, ssf), 0)

    ra0 = prep_fire(jnp.int32(0), sa)

    def pair_body(s, ra):
        rb = prep_fire(2 * s + 1, sb)
        drain(sa, ra)
        ra_next = prep_fire(2 * s + 2, sa)
        drain(sb, rb)
        return ra_next

    ra_last = lax.fori_loop(0, _NPAIR, pair_body, ra0)
    drain(sa, ra_last)


def _set_scratch():
    return [
        pltpu.VMEM((_P + 2 * _G,), jnp.int32),
        pltpu.VMEM((_P + 2 * _G,), jnp.int32),
        pltpu.VMEM((_P + 2 * _G,), jnp.int32),
        pltpu.VMEM((_NR + 1, _G), jnp.int32),
        pltpu.VMEM((_NR + 1, _G), jnp.int32),
        pltpu.VMEM((_NR + 1, _G), jnp.int32),
        pltpu.VMEM((_P + 3 * _G, _D), jnp.float32),
        pltpu.SemaphoreType.DMA,
        pltpu.SemaphoreType.DMA,
        pltpu.SemaphoreType.DMA,
        pltpu.SemaphoreType.DMA,
        pltpu.SemaphoreType.DMA,
        pltpu.SemaphoreType.DMA,
    ]


_launch = pl.kernel(
    _body,
    out_type=jax.ShapeDtypeStruct((_NPOS, _D), jnp.float32),
    compiler_params=pltpu.CompilerParams(needs_layout_passes=False),
    mesh=plsc.VectorSubcoreMesh(core_axis_name="c", subcore_axis_name="s"),
    scratch_types=[
        pltpu.VMEM((_PW,), jnp.int32),
        pltpu.VMEM((_PW,), jnp.int32),
        pltpu.VMEM((16,), jnp.int32),
    ] + _set_scratch() + _set_scratch(),
)


def kernel(caption_indices, entities_encoded, facts_encoded, word_embedding,
           pad_token, caption_masks):
    cap = caption_indices.reshape(_NPOS)
    msk = caption_masks.reshape(_NPOS)
    ents = entities_encoded.reshape(_B * _NE, _D)
    facts = facts_encoded.reshape(_B * _NF, _D)
    pad = jnp.full((16,), pad_token, dtype=jnp.int32)
    out = _launch(cap, msk, ents, facts, word_embedding, pad)
    return out.reshape(_B, _L, _D)


# speedup vs baseline: 1.0988x; 1.0988x over previous
import jax
import jax.numpy as jnp
from jax import lax
from jax.experimental import pallas as pl
from jax.experimental.pallas import tpu as pltpu
from jax.experimental.pallas import tpu_sc as plsc

_B, _L, _D = 4096, 50, 128
_VOCAB = 100000
_NE = 32
_NF = 32
_NPOS = _B * _L
_NWORK = 32
_PW = _NPOS // _NWORK
_P = 256
_NCHUNK = _PW // _P
_NPAIR = (_NCHUNK - 1) // 2
_NG = _P // 16
_G = 8
_GSH = 3
_NR = _P // _G


def _body(cidx_hbm, msk_hbm, ents_hbm, facts_hbm, word_hbm, pad_hbm, out_hbm,
          cidx_v, msk_v, pad_v, *sets):
    sa, sb = sets[:13], sets[13:]
    wid = lax.axis_index("s") * 2 + lax.axis_index("c")
    wbase = wid * _PW
    lane = jnp.arange(16, dtype=jnp.int32)
    c50 = jnp.full((16,), 50, dtype=jnp.int32)

    pltpu.sync_copy(cidx_hbm.at[pl.ds(wbase, _PW)], cidx_v)
    pltpu.sync_copy(msk_hbm.at[pl.ds(wbase, _PW)], msk_v)
    pltpu.sync_copy(pad_hbm, pad_v)
    padv = pad_v[...]

    def compact(srcl, dst2, cnt, srcv, posv, m):
        s = plsc.cumsum(jnp.where(m, 1, 0).astype(jnp.int32))
        lin = cnt + s - 1
        plsc.store_scatter(srcl, [lin], srcv, mask=m)
        plsc.store_scatter(dst2, [lin >> _GSH, lin & (_G - 1)], posv, mask=m)
        return cnt + jnp.sum(jnp.where(m, 1, 0).astype(jnp.int32))

    def pad_tail(srcl, dst2, cnt):
        last = jnp.maximum(cnt - 1, 0)
        lv = plsc.load_gather(srcl, [jnp.full((16,), last, dtype=jnp.int32)])
        dv = plsc.load_gather(
            dst2, [jnp.full((16,), last >> _GSH, dtype=jnp.int32),
                   jnp.full((16,), last & (_G - 1), dtype=jnp.int32)])
        for o in range(0, max(_G, 16), 16):
            lin = cnt + o + lane
            plsc.store_scatter(srcl, [lin], lv)
            plsc.store_scatter(dst2, [lin >> _GSH, lin & (_G - 1)], dv)

    def prep_fire(c, S):
        (srcw, srce, srcf, dstw, dste, dstf, rows,
         semw, seme, semf, _ssw, _sse, _ssf) = S
        cbase = c * _P
        cw = jnp.int32(0)
        ce = jnp.int32(0)
        cf = jnp.int32(0)
        for g in range(_NG):
            off = cbase + g * 16
            iv = cidx_v[pl.ds(off, 16)]
            mv = msk_v[pl.ds(off, 16)]
            pos = (wbase + off) + lane
            rowb = lax.div(pos, c50) * _NE
            te = iv - _VOCAB
            cei = jnp.where((te < 0) | (te >= _NE), _NE - 1, te)
            tf = te - _NE
            cfi = jnp.where((tf < 0) | (tf >= _NF), _NF - 1, tf)
            wv = jnp.where(iv >= _VOCAB, padv, iv)
            m_e = mv == 1
            m_f = mv == 2
            m_w = ~(m_e | m_f)
            cw = compact(srcw, dstw, cw, wv, pos, m_w)
            ce = compact(srce, dste, ce, rowb + cei, pos, m_e)
            cf = compact(srcf, dstf, cf, rowb + cfi, pos, m_f)

        pad_tail(srcw, dstw, cw)
        pad_tail(srce, dste, ce)
        pad_tail(srcf, dstf, cf)

        rw = (cw + _G - 1) >> _GSH
        re_ = (ce + _G - 1) >> _GSH
        rf = (cf + _G - 1) >> _GSH

        def fire(tab, srcl, base, sem):
            def go(r, _):
                pltpu.async_copy(tab.at[srcl.at[pl.ds(r * _G, _G)]],
                                 rows.at[pl.ds(base + r * _G, _G)], sem)
                return 0
            return go

        lax.fori_loop(0, rw, fire(word_hbm, srcw, jnp.int32(0), semw), 0)
        lax.fori_loop(0, re_, fire(ents_hbm, srce, rw * _G, seme), 0)
        lax.fori_loop(0, rf, fire(facts_hbm, srcf, (rw + re_) * _G, semf), 0)
        return rw, re_, rf

    def drain(S, r3):
        (srcw, srce, srcf, dstw, dste, dstf, rows,
         semw, seme, semf, ssw, sse, ssf) = S
        rw, re_, rf = r3
        bw = jnp.int32(0)
        be = rw * _G
        bf = (rw + re_) * _G

        def wait_gather(tab, srcl, base, sem):
            def go(r, _):
                pltpu.make_async_copy(tab.at[srcl.at[pl.ds(r * _G, _G)]],
                                      rows.at[pl.ds(base + r * _G, _G)],
                                      sem).wait()
                return 0
            return go

        def scat(dst2, base, sem):
            def go(r, _):
                pltpu.async_copy(rows.at[pl.ds(base + r * _G, _G)],
                                 out_hbm.at[dst2.at[r]], sem)
                return 0
            return go

        def wait_scat(dst2, base, sem):
            def go(r, _):
                pltpu.make_async_copy(rows.at[pl.ds(base + r * _G, _G)],
                                      out_hbm.at[dst2.at[r]], sem).wait()
                return 0
            return go

        lax.fori_loop(0, rw, wait_gather(word_hbm, srcw, bw, semw), 0)
        lax.fori_loop(0, rw, scat(dstw, bw, ssw), 0)
        lax.fori_loop(0, re_, wait_gather(ents_hbm, srce, be, seme), 0)
        lax.fori_loop(0, re_, scat(dste, be, sse), 0)
        lax.fori_loop(0, rf, wait_gather(facts_hbm, srcf, bf, semf), 0)
        lax.fori_loop(0, rf, scat(dstf, bf, ssf), 0)
        lax.fori_loop(0, rw, wait_scat(dstw, bw, ssw), 0)
        lax.fori_loop(0, re_, wait_scat(dste, be, sse), 0)
        lax.fori_loop(0, rf, wait_scat(dstf, bf, ssf), 0)

    ra0 = prep_fire(jnp.int32(0), sa)

    def pair_body(s, ra):
        rb = prep_fire(2 * s + 1, sb)
        drain(sa, ra)
        ra_next = prep_fire(2 * s + 2, sa)
        drain(sb, rb)
        return ra_next

    ra_last = lax.fori_loop(0, _NPAIR, pair_body, ra0)
    drain(sa, ra_last)


def _set_scratch():
    return [
        pltpu.VMEM((_P + 4 * _G + 16,), jnp.int32),
        pltpu.VMEM((_P + 4 * _G + 16,), jnp.int32),
        pltpu.VMEM((_P + 4 * _G + 16,), jnp.int32),
        pltpu.VMEM(((_P + 2 * _G + 16) // _G + 1, _G), jnp.int32),
        pltpu.VMEM(((_P + 2 * _G + 16) // _G + 1, _G), jnp.int32),
        pltpu.VMEM(((_P + 2 * _G + 16) // _G + 1, _G), jnp.int32),
        pltpu.VMEM((_P + 3 * _G, _D), jnp.float32),
        pltpu.SemaphoreType.DMA,
        pltpu.SemaphoreType.DMA,
        pltpu.SemaphoreType.DMA,
        pltpu.SemaphoreType.DMA,
        pltpu.SemaphoreType.DMA,
        pltpu.SemaphoreType.DMA,
    ]


_launch = pl.kernel(
    _body,
    out_type=jax.ShapeDtypeStruct((_NPOS, _D), jnp.float32),
    compiler_params=pltpu.CompilerParams(needs_layout_passes=False),
    mesh=plsc.VectorSubcoreMesh(core_axis_name="c", subcore_axis_name="s"),
    scratch_types=[
        pltpu.VMEM((_PW,), jnp.int32),
        pltpu.VMEM((_PW,), jnp.int32),
        pltpu.VMEM((16,), jnp.int32),
    ] + _set_scratch() + _set_scratch(),
)


def kernel(caption_indices, entities_encoded, facts_encoded, word_embedding,
           pad_token, caption_masks):
    cap = caption_indices.reshape(_NPOS)
    msk = caption_masks.reshape(_NPOS)
    ents = entities_encoded.reshape(_B * _NE, _D)
    facts = facts_encoded.reshape(_B * _NF, _D)
    pad = jnp.full((16,), pad_token, dtype=jnp.int32)
    out = _launch(cap, msk, ents, facts, word_embedding, pad)
    return out.reshape(_B, _L, _D)
